# baseline (speedup 1.0000x reference)
import jax
import jax.numpy as jnp
from jax import lax
from jax.experimental import pallas as pl
from jax.experimental.pallas import tpu as pltpu
from jax.experimental.pallas import tpu_sc as plsc

B = 2
N_BOXES = 1000
C = 256
OUT = 7
CELLS = OUT * OUT
ROWS_PER_BATCH = 21845 - (21845 >> 10)
TOTAL_BOXES = B * N_BOXES

NC = 2
NS = 16
NW = NC * NS
CHUNK = -(-TOTAL_BOXES // NW)
BSTAGE = CHUNK * 4 + 8

HALF_BASE = (0, 24)
HALF_CELLS = (24, 25)
HALF_GROWS = (24, 32)


def _sc_body(f2, f3, f4, f5, f6, boxes_hbm, out_hbm, *s):
    tables = (f2, f3, f4, f5, f6)
    (boxes_v, yr0_v, yr1_v, xi0_v, xi1_v, wy0_v, wy1_v, wx0_v, wx1_v,
     out_v) = s[:10]
    bufs = (s[10:23], s[23:36])

    wid = lax.axis_index("s") * NC + lax.axis_index("c")
    lo = wid * CHUNK
    hi = jnp.minimum(lo + CHUNK, TOTAL_BOXES)
    astart = pl.multiple_of((lo * 4) - ((lo * 4) % 8), 8)

    pltpu.sync_copy(boxes_hbm.at[pl.ds(astart, BSTAGE)], boxes_v)

    iota = lax.broadcasted_iota(jnp.int32, (16,), 0)
    iota_f = iota.astype(jnp.float32)

    def build_params(n):
        bb = lax.broadcast(n * 4 - astart, (16,))
        y1 = plsc.load_gather(boxes_v, [bb])
        x1 = plsc.load_gather(boxes_v, [bb + 1])
        y2 = plsc.load_gather(boxes_v, [bb + 2])
        x2 = plsc.load_gather(boxes_v, [bb + 3])

        h = y2 - y1
        w = x2 - x1
        area = h * w
        lvl = (jnp.int32(2)
               + jnp.where(area >= 12544.0, 1, 0)
               + jnp.where(area >= 50176.0, 1, 0)
               + jnp.where(area >= 200704.0, 1, 0)
               + jnp.where(area >= 802816.0, 1, 0))
        scale = lax.shift_left(jnp.int32(1), lvl).astype(jnp.float32)
        l2 = lvl - 2
        wl_i = lax.shift_right_logical(jnp.int32(128), l2)
        wl_f = wl_i.astype(jnp.float32)
        batch = n // N_BOXES
        base = lax.broadcast(batch, (16,)) * wl_i * wl_i

        bh = h / scale
        bw = w / scale
        gy = y1 / scale + (iota_f + 0.5) * bh / 7.0
        gx = x1 / scale + (iota_f + 0.5) * bw / 7.0
        bnd = wl_f - 1.0

        y0f = jnp.maximum(gy.astype(jnp.int32).astype(jnp.float32), 0.0)
        ly = gy - y0f
        yi0 = jnp.minimum(y0f, bnd).astype(jnp.int32)
        yi1 = jnp.minimum(y0f + 1.0, bnd).astype(jnp.int32)
        x0f = jnp.maximum(gx.astype(jnp.int32).astype(jnp.float32), 0.0)
        lx = gx - x0f
        xi0 = jnp.minimum(x0f, bnd).astype(jnp.int32)
        xi1 = jnp.minimum(x0f + 1.0, bnd).astype(jnp.int32)

        yr0_v[...] = base + yi0 * wl_i
        yr1_v[...] = base + yi1 * wl_i
        xi0_v[...] = xi0
        xi1_v[...] = xi1
        wy0_v[...] = 1.0 - ly
        wy1_v[...] = ly
        wx0_v[...] = 1.0 - lx
        wx1_v[...] = lx
        return lax.reduce_max(l2, axes=(0,))

    def fire_half(half, buf, lsc):
        (i00, i01, i10, i11, w00, w01, w10, w11, sem) = buf[:9]
        r00, r01, r10, r11 = buf[9:]
        cbase = HALF_BASE[half]
        for st in (0, 16):
            c = iota + (cbase + st)
            i = c // 7
            j = c - i * 7
            gy0 = plsc.load_gather(yr0_v, [i])
            gy1 = plsc.load_gather(yr1_v, [i])
            gx0 = plsc.load_gather(xi0_v, [j])
            gx1 = plsc.load_gather(xi1_v, [j])
            vwy0 = plsc.load_gather(wy0_v, [i])
            vwy1 = plsc.load_gather(wy1_v, [i])
            vwx0 = plsc.load_gather(wx0_v, [j])
            vwx1 = plsc.load_gather(wx1_v, [j])
            ssl = pl.ds(st, 16)
            i00[ssl] = gy0 + gx0
            i01[ssl] = gy0 + gx1
            i10[ssl] = gy1 + gx0
            i11[ssl] = gy1 + gx1
            w00[ssl] = vwy0 * vwx0
            w01[ssl] = vwy0 * vwx1
            w10[ssl] = vwy1 * vwx0
            w11[ssl] = vwy1 * vwx1

        gsl = pl.ds(0, HALF_GROWS[half])
        for k in range(5):
            @pl.when(lsc == k)
            def _():
                ft = tables[k]
                pltpu.async_copy(ft.at[i00.at[gsl]], r00.at[gsl], sem)
                pltpu.async_copy(ft.at[i01.at[gsl]], r01.at[gsl], sem)
                pltpu.async_copy(ft.at[i10.at[gsl]], r10.at[gsl], sem)
                pltpu.async_copy(ft.at[i11.at[gsl]], r11.at[gsl], sem)

    def compute_half(n, half, buf):
        (i00, i01, i10, i11, w00, w01, w10, w11, sem) = buf[:9]
        r00, r01, r10, r11 = buf[9:]
        gsl = pl.ds(0, HALF_GROWS[half])
        pltpu.make_async_copy(f2.at[i00.at[gsl]], r00.at[gsl], sem).wait()
        pltpu.make_async_copy(f2.at[i01.at[gsl]], r01.at[gsl], sem).wait()
        pltpu.make_async_copy(f2.at[i10.at[gsl]], r10.at[gsl], sem).wait()
        pltpu.make_async_copy(f2.at[i11.at[gsl]], r11.at[gsl], sem).wait()
        cbase = HALF_BASE[half]

        def per_cell(cell, _):
            g = cbase + cell
            gi = g // 7
            gj = g - gi * 7
            cs = lax.broadcast(cell, (16,))
            a00 = plsc.load_gather(w00, [cs])
            a01 = plsc.load_gather(w01, [cs])
            a10 = plsc.load_gather(w10, [cs])
            a11 = plsc.load_gather(w11, [cs])
            for t in range(C // 16):
                cc = pl.ds(t * 16, 16)
                out_v[gi, gj, cc] = (
                    a00 * r00[cell, cc] + a01 * r01[cell, cc]
                    + a10 * r10[cell, cc] + a11 * r11[cell, cc])
            return 0

        lax.fori_loop(0, HALF_CELLS[half], per_cell, 0)

        if half == 1:
            bt = n // N_BOXES
            pltpu.sync_copy(out_v, out_hbm.at[bt, n - bt * N_BOXES])

    ba, bb_ = bufs
    l0 = build_params(lo)
    fire_half(0, ba, l0)
    fire_half(1, bb_, l0)

    def step(n, _):
        compute_half(n, 0, ba)
        lnext = build_params(n + 1)

        @pl.when(n + 1 < hi)
        def _():
            fire_half(0, ba, lnext)

        compute_half(n, 1, bb_)

        @pl.when(n + 1 < hi)
        def _():
            fire_half(1, bb_, lnext)

        return 0

    lax.fori_loop(lo, hi, step, 0)


def kernel(feat_l2, feat_l3, feat_l4, feat_l5, feat_l6, boxes):
    feats = [f.reshape(-1, C) for f in
             (feat_l2, feat_l3, feat_l4, feat_l5, feat_l6)]
    boxes_flat = jnp.pad(boxes.reshape(TOTAL_BOXES * 4), (0, 72))

    def buf_types():
        return ([pltpu.VMEM((32,), jnp.int32) for _ in range(4)]
                + [pltpu.VMEM((32,), jnp.float32) for _ in range(4)]
                + [pltpu.SemaphoreType.DMA]
                + [pltpu.VMEM((32, C), jnp.float32) for _ in range(4)])

    mesh = plsc.VectorSubcoreMesh(core_axis_name="c", subcore_axis_name="s")
    run = pl.kernel(
        _sc_body,
        mesh=mesh,
        compiler_params=pltpu.CompilerParams(needs_layout_passes=False),
        out_type=jax.ShapeDtypeStruct((B, N_BOXES, OUT, OUT, C), jnp.float32),
        scratch_types=(
            [pltpu.VMEM((BSTAGE,), jnp.float32)]
            + [pltpu.VMEM((16,), jnp.int32) for _ in range(4)]
            + [pltpu.VMEM((16,), jnp.float32) for _ in range(4)]
            + [pltpu.VMEM((OUT, OUT, C), jnp.float32)]
            + buf_types() + buf_types()
        ),
    )
    return run(*feats, boxes_flat)


# speedup vs baseline: 1.3313x; 1.3313x over previous
import jax
import jax.numpy as jnp
from jax import lax
from jax.experimental import pallas as pl
from jax.experimental.pallas import tpu as pltpu
from jax.experimental.pallas import tpu_sc as plsc

B = 2
N_BOXES = 1000
C = 256
OUT = 7
CELLS = OUT * OUT
ROWS_PER_BATCH = 21845 - (21845 >> 10)
TOTAL_BOXES = B * N_BOXES

NC = 2
NS = 16
NW = NC * NS
CHUNK = -(-TOTAL_BOXES // NW)
BSTAGE = CHUNK * 4 + 8

HALF_BASE = (0, 24)
HALF_CELLS = (24, 25)
HALF_GROWS = (24, 32)


def _sc_body(f2, f3, f4, f5, f6, boxes_hbm, out_hbm, *s):
    tables = (f2, f3, f4, f5, f6)
    (boxes_v, yr0_v, yr1_v, xi0_v, xi1_v, wy0_v, wy1_v, wx0_v, wx1_v,
     out_v) = s[:10]
    bufs = (s[10:23], s[23:36])

    wid = lax.axis_index("s") * NC + lax.axis_index("c")
    lo = wid * CHUNK
    hi = jnp.minimum(lo + CHUNK, TOTAL_BOXES)
    astart = pl.multiple_of((lo * 4) - ((lo * 4) % 8), 8)

    pltpu.sync_copy(boxes_hbm.at[pl.ds(astart, BSTAGE)], boxes_v)

    iota = lax.broadcasted_iota(jnp.int32, (16,), 0)
    iota_f = iota.astype(jnp.float32)

    def build_params(n):
        bb = lax.broadcast(n * 4 - astart, (16,))
        y1 = plsc.load_gather(boxes_v, [bb])
        x1 = plsc.load_gather(boxes_v, [bb + 1])
        y2 = plsc.load_gather(boxes_v, [bb + 2])
        x2 = plsc.load_gather(boxes_v, [bb + 3])

        h = y2 - y1
        w = x2 - x1
        area = h * w
        lvl = (jnp.int32(2)
               + jnp.where(area >= 12544.0, 1, 0)
               + jnp.where(area >= 50176.0, 1, 0)
               + jnp.where(area >= 200704.0, 1, 0)
               + jnp.where(area >= 802816.0, 1, 0))
        scale = lax.shift_left(jnp.int32(1), lvl).astype(jnp.float32)
        l2 = lvl - 2
        wl_i = lax.shift_right_logical(jnp.int32(128), l2)
        wl_f = wl_i.astype(jnp.float32)
        batch = n // N_BOXES
        base = lax.broadcast(batch, (16,)) * wl_i * wl_i

        bh = h / scale
        bw = w / scale
        gy = y1 / scale + (iota_f + 0.5) * bh / 7.0
        gx = x1 / scale + (iota_f + 0.5) * bw / 7.0
        bnd = wl_f - 1.0

        y0f = jnp.maximum(gy.astype(jnp.int32).astype(jnp.float32), 0.0)
        ly = gy - y0f
        yi0 = jnp.minimum(y0f, bnd).astype(jnp.int32)
        yi1 = jnp.minimum(y0f + 1.0, bnd).astype(jnp.int32)
        x0f = jnp.maximum(gx.astype(jnp.int32).astype(jnp.float32), 0.0)
        lx = gx - x0f
        xi0 = jnp.minimum(x0f, bnd).astype(jnp.int32)
        xi1 = jnp.minimum(x0f + 1.0, bnd).astype(jnp.int32)

        yr0_v[...] = base + yi0 * wl_i
        yr1_v[...] = base + yi1 * wl_i
        xi0_v[...] = xi0
        xi1_v[...] = xi1
        wy0_v[...] = 1.0 - ly
        wy1_v[...] = ly
        wx0_v[...] = 1.0 - lx
        wx1_v[...] = lx
        return lax.reduce_max(l2, axes=(0,))

    def fire_half(half, buf, lsc):
        (i00, i01, i10, i11, w00, w01, w10, w11, sem) = buf[:9]
        r00, r01, r10, r11 = buf[9:]
        cbase = HALF_BASE[half]
        for st in (0, 16):
            c = iota + (cbase + st)
            i = c // 7
            j = c - i * 7
            gy0 = plsc.load_gather(yr0_v, [i])
            gy1 = plsc.load_gather(yr1_v, [i])
            gx0 = plsc.load_gather(xi0_v, [j])
            gx1 = plsc.load_gather(xi1_v, [j])
            vwy0 = plsc.load_gather(wy0_v, [i])
            vwy1 = plsc.load_gather(wy1_v, [i])
            vwx0 = plsc.load_gather(wx0_v, [j])
            vwx1 = plsc.load_gather(wx1_v, [j])
            ssl = pl.ds(st, 16)
            i00[ssl] = gy0 + gx0
            i01[ssl] = gy0 + gx1
            i10[ssl] = gy1 + gx0
            i11[ssl] = gy1 + gx1
            w00[ssl] = vwy0 * vwx0
            w01[ssl] = vwy0 * vwx1
            w10[ssl] = vwy1 * vwx0
            w11[ssl] = vwy1 * vwx1

        gsl = pl.ds(0, HALF_GROWS[half])
        for k in range(5):
            @pl.when(lsc == k)
            def _():
                ft = tables[k]
                pltpu.async_copy(ft.at[i00.at[gsl]], r00.at[gsl], sem)
                pltpu.async_copy(ft.at[i01.at[gsl]], r01.at[gsl], sem)
                pltpu.async_copy(ft.at[i10.at[gsl]], r10.at[gsl], sem)
                pltpu.async_copy(ft.at[i11.at[gsl]], r11.at[gsl], sem)

    def compute_half(n, half, buf):
        (i00, i01, i10, i11, w00, w01, w10, w11, sem) = buf[:9]
        r00, r01, r10, r11 = buf[9:]
        gsl = pl.ds(0, HALF_GROWS[half])
        pltpu.make_async_copy(f2.at[i00.at[gsl]], r00.at[gsl], sem).wait()
        pltpu.make_async_copy(f2.at[i01.at[gsl]], r01.at[gsl], sem).wait()
        pltpu.make_async_copy(f2.at[i10.at[gsl]], r10.at[gsl], sem).wait()
        pltpu.make_async_copy(f2.at[i11.at[gsl]], r11.at[gsl], sem).wait()
        cbase = HALF_BASE[half]

        def per_cell(cell, _):
            cs = lax.broadcast(cell, (16,))
            a00 = plsc.load_gather(w00, [cs])
            a01 = plsc.load_gather(w01, [cs])
            a10 = plsc.load_gather(w10, [cs])
            a11 = plsc.load_gather(w11, [cs])
            for t in range(C // 16):
                cc = pl.ds(t * 16, 16)
                out_v[cbase + cell, cc] = (
                    a00 * r00[cell, cc] + a01 * r01[cell, cc]
                    + a10 * r10[cell, cc] + a11 * r11[cell, cc])
            return 0

        lax.fori_loop(0, HALF_CELLS[half], per_cell, 0)

        if half == 1:
            pltpu.sync_copy(out_v, out_hbm.at[n])

    ba, bb_ = bufs
    l0 = build_params(lo)
    fire_half(0, ba, l0)
    fire_half(1, bb_, l0)

    def step(n, _):
        compute_half(n, 0, ba)
        lnext = build_params(n + 1)

        @pl.when(n + 1 < hi)
        def _():
            fire_half(0, ba, lnext)

        compute_half(n, 1, bb_)

        @pl.when(n + 1 < hi)
        def _():
            fire_half(1, bb_, lnext)

        return 0

    lax.fori_loop(lo, hi, step, 0)


def kernel(feat_l2, feat_l3, feat_l4, feat_l5, feat_l6, boxes):
    feats = [f.reshape(-1, C) for f in
             (feat_l2, feat_l3, feat_l4, feat_l5, feat_l6)]
    boxes_flat = jnp.pad(boxes.reshape(TOTAL_BOXES * 4), (0, 72))

    def buf_types():
        return ([pltpu.VMEM((32,), jnp.int32) for _ in range(4)]
                + [pltpu.VMEM((32,), jnp.float32) for _ in range(4)]
                + [pltpu.SemaphoreType.DMA]
                + [pltpu.VMEM((32, C), jnp.float32) for _ in range(4)])

    mesh = plsc.VectorSubcoreMesh(core_axis_name="c", subcore_axis_name="s")
    run = pl.kernel(
        _sc_body,
        mesh=mesh,
        compiler_params=pltpu.CompilerParams(needs_layout_passes=False),
        out_type=jax.ShapeDtypeStruct((TOTAL_BOXES, CELLS, C), jnp.float32),
        scratch_types=(
            [pltpu.VMEM((BSTAGE,), jnp.float32)]
            + [pltpu.VMEM((16,), jnp.int32) for _ in range(4)]
            + [pltpu.VMEM((16,), jnp.float32) for _ in range(4)]
            + [pltpu.VMEM((CELLS, C), jnp.float32)]
            + buf_types() + buf_types()
        ),
    )
    out = run(*feats, boxes_flat)
    return out.reshape(B, N_BOXES, OUT, OUT, C)
